# baseline (speedup 1.0000x reference)
import functools

import jax
import jax.numpy as jnp
from jax import lax
from jax.experimental import pallas as pl
from jax.experimental.pallas import tpu as pltpu
from jax.experimental.pallas import tpu_sc as plsc

N = 10000
E = 160000
HID = 32
NGRAPH = 64

NC = 2
NS = 16
NW = NC * NS
CH = 128
NCH = 40
EW = CH * NCH
EPAD = NW * EW
D = 32

@functools.lru_cache(maxsize=None)
def _sc_mesh():
    return plsc.VectorSubcoreMesh(core_axis_name="c", subcore_axis_name="s",
                                  num_cores=NC, num_subcores=NS)


def _gather_body(x_hbm, idx_hbm, out_hbm, idx_v, rows_v, sem):
    cid = lax.axis_index("c")
    sid = lax.axis_index("s")
    wid = sid * NC + cid
    pltpu.sync_copy(idx_hbm.at[pl.ds(wid * NCH, NCH)], idx_v)

    def step(j, carry):
        pltpu.async_copy(x_hbm.at[idx_v.at[j]], rows_v, sem).wait()
        pltpu.sync_copy(rows_v, out_hbm.at[pl.ds(wid * EW + j * CH, CH)])
        return carry

    lax.fori_loop(0, NCH, step, 0)


@functools.lru_cache(maxsize=None)
def _gather_call():
    return pl.kernel(
        _gather_body,
        out_type=jax.ShapeDtypeStruct((EPAD, D), jnp.float32),
        mesh=_sc_mesh(),
        compiler_params=pltpu.CompilerParams(use_tc_tiling_on_sc=False),
        scratch_types=[
            pltpu.VMEM((NCH, CH), jnp.int32),
            pltpu.VMEM((CH, D), jnp.float32),
            pltpu.SemaphoreType.DMA,
        ],
    )


def _sc_gather(x, idx):
    return _gather_call()(x, idx)


def _scatter_body(msg_hbm, idx_hbm, zero_hbm, out_hbm, idx_v, buf_v, acc_sh, sem):
    cid = lax.axis_index("c")
    sid = lax.axis_index("s")
    wid = sid * NC + cid

    @pl.when(sid == 0)
    def _():
        pltpu.sync_copy(zero_hbm, acc_sh)

    plsc.subcore_barrier()
    pltpu.sync_copy(idx_hbm.at[pl.ds(wid * NCH, NCH)], idx_v)

    def step(j, carry):
        pltpu.async_copy(msg_hbm.at[pl.ds(wid * EW + j * CH, CH)], buf_v, sem).wait()
        pltpu.sync_copy(buf_v, acc_sh.at[idx_v.at[j]], add=True)
        return carry

    lax.fori_loop(0, NCH, step, 0)
    plsc.subcore_barrier()
    rb = N // NS
    pltpu.sync_copy(acc_sh.at[pl.ds(sid * rb, rb)],
                    out_hbm.at[pl.ds(cid * N + sid * rb, rb)])


@functools.lru_cache(maxsize=None)
def _scatter_call():
    return pl.kernel(
        _scatter_body,
        out_type=jax.ShapeDtypeStruct((NC * N, D), jnp.float32),
        mesh=_sc_mesh(),
        compiler_params=pltpu.CompilerParams(use_tc_tiling_on_sc=False),
        scratch_types=[
            pltpu.VMEM((NCH, CH), jnp.int32),
            pltpu.VMEM((CH, D), jnp.float32),
            pltpu.VMEM_SHARED((N, D), jnp.float32),
            pltpu.SemaphoreType.DMA,
        ],
    )


def _sc_scatter(msg, idx, zeros):
    return _scatter_call()(msg, idx, zeros)


BE = 512


def _msg_body(c_real, ea_ref, xs_ref, w1t_ref, b1_ref, v_ref, b2_ref, out_ref):
    pid = pl.program_id(0)
    h = jnp.dot(ea_ref[...], w1t_ref[...], preferred_element_type=jnp.float32)
    h = jnp.maximum(h + b1_ref[...], 0.0)
    xs = xs_ref[...]
    u = jnp.concatenate([xs[:, i:i + 1] * h for i in range(c_real)], axis=1)
    msg = jnp.dot(u.astype(jnp.bfloat16), v_ref[...],
                  preferred_element_type=jnp.float32)
    msg = msg + jnp.dot(xs, b2_ref[...], preferred_element_type=jnp.float32)
    gid = pid * BE + lax.broadcasted_iota(jnp.int32, (BE, HID), 0)
    out_ref[...] = jnp.where(gid < E, msg, 0.0)


def _make_msg(c_real):
    return pl.pallas_call(
        functools.partial(_msg_body, c_real),
        grid=(EPAD // BE,),
        in_specs=[
            pl.BlockSpec((BE, 8), lambda i: (i, 0)),
            pl.BlockSpec((BE, D), lambda i: (i, 0)),
            pl.BlockSpec((8, 128), lambda i: (0, 0)),
            pl.BlockSpec((1, 128), lambda i: (0, 0)),
            pl.BlockSpec((c_real * 128, HID), lambda i: (0, 0)),
            pl.BlockSpec((D, HID), lambda i: (0, 0)),
        ],
        out_specs=pl.BlockSpec((BE, HID), lambda i: (i, 0)),
        out_shape=jax.ShapeDtypeStruct((EPAD, HID), jnp.float32),
    )


BN = 2000


def _pointwise_body(act, p0, p1, c0, c1, x_ref, rt_ref, cb_ref, g_ref, b_ref, out_ref):
    s = p0[...] + p1[...]
    cnt = c0[...][:, :1] + c1[...][:, :1]
    mean = s / jnp.maximum(cnt, 1.0)
    r = jnp.dot(x_ref[...], rt_ref[...], preferred_element_type=jnp.float32)
    t = mean + r + cb_ref[...]
    m = jnp.mean(t, axis=1, keepdims=True)
    d = t - m
    v = jnp.mean(d * d, axis=1, keepdims=True)
    y = d * lax.rsqrt(v + 1e-5) * g_ref[...] + b_ref[...]
    if act == "relu":
        y = jnp.maximum(y, 0.0)
    elif act == "elu":
        y = jnp.where(y > 0.0, y, jnp.exp(jnp.minimum(y, 0.0)) - 1.0)
    else:
        y = jnp.where(y > 0.0, y, 0.01 * y)
    out_ref[...] = y


def _make_pointwise(act):
    blk = lambda shape: pl.BlockSpec(shape, lambda i: (i, 0))
    full = lambda shape: pl.BlockSpec(shape, lambda i: (0, 0))
    return pl.pallas_call(
        functools.partial(_pointwise_body, act),
        grid=(N // BN,),
        in_specs=[
            blk((BN, HID)), blk((BN, HID)), blk((BN, D)), blk((BN, D)),
            blk((BN, D)), full((D, HID)), full((1, HID)), full((1, HID)),
            full((1, HID)),
        ],
        out_specs=blk((BN, HID)),
        out_shape=jax.ShapeDtypeStruct((N, HID), jnp.float32),
    )


BP = 400


def _pool_body(h_ref, bt_ref, ct_ref, ld_ref, wm_ref, wx_ref, wc_ref, wl_ref,
               lb_ref, out_ref, acc_s, acc_c, acc_m):
    pid = pl.program_id(0)

    @pl.when(pid == 0)
    def _():
        acc_s[...] = jnp.zeros((NGRAPH, HID), jnp.float32)
        acc_c[...] = jnp.zeros((NGRAPH, HID), jnp.float32)
        acc_m[...] = jnp.full((NGRAPH, HID), -jnp.inf, jnp.float32)

    h = h_ref[...]
    bt = bt_ref[...][:, :1]
    ohb = bt == lax.broadcasted_iota(jnp.int32, (BP, NGRAPH), 1)
    oh = ohb.astype(jnp.float32)
    dn = (((0,), (0,)), ((), ()))
    acc_s[...] += lax.dot_general(oh, h, dn, preferred_element_type=jnp.float32)
    acc_c[...] += lax.dot_general(oh, jnp.ones((BP, HID), jnp.float32), dn,
                                  preferred_element_type=jnp.float32)
    mx = jnp.concatenate(
        [jnp.max(jnp.where(bt == g, h, -jnp.inf), axis=0, keepdims=True)
         for g in range(NGRAPH)], axis=0)
    acc_m[...] = jnp.maximum(acc_m[...], mx)

    mean = acc_s[...] / jnp.maximum(acc_c[...], 1.0)
    out = jnp.dot(mean, wm_ref[...], preferred_element_type=jnp.float32)
    out += jnp.dot(acc_m[...], wx_ref[...], preferred_element_type=jnp.float32)
    out += jnp.dot(ct_ref[...], wc_ref[...], preferred_element_type=jnp.float32)
    out += jnp.dot(ld_ref[...], wl_ref[...], preferred_element_type=jnp.float32)
    out_ref[...] = out + lb_ref[...]


_pool = pl.pallas_call(
    _pool_body,
    grid=(N // BP,),
    in_specs=[
        pl.BlockSpec((BP, HID), lambda i: (i, 0)),
        pl.BlockSpec((BP, 8), lambda i: (i, 0)),
        pl.BlockSpec((NGRAPH, 4), lambda i: (0, 0)),
        pl.BlockSpec((NGRAPH, 1), lambda i: (0, 0)),
        pl.BlockSpec((HID, 2), lambda i: (0, 0)),
        pl.BlockSpec((HID, 2), lambda i: (0, 0)),
        pl.BlockSpec((4, 2), lambda i: (0, 0)),
        pl.BlockSpec((1, 2), lambda i: (0, 0)),
        pl.BlockSpec((1, 2), lambda i: (0, 0)),
    ],
    out_specs=pl.BlockSpec((NGRAPH, 2), lambda i: (0, 0)),
    out_shape=jax.ShapeDtypeStruct((NGRAPH, 2), jnp.float32),
    scratch_shapes=[
        pltpu.VMEM((NGRAPH, HID), jnp.float32),
        pltpu.VMEM((NGRAPH, HID), jnp.float32),
        pltpu.VMEM((NGRAPH, HID), jnp.float32),
    ],
)


def kernel(x, edge_index, edge_attr, batch, contingency_type, load_scale, params):
    p = params
    f32 = jnp.float32
    src, dst = edge_index[0], edge_index[1]
    pad = EPAD - E
    zpad = jnp.zeros((pad,), jnp.int32)
    src_p = jnp.concatenate([src, zpad]).reshape(EPAD // CH, CH)
    dst_p = jnp.concatenate([dst, zpad]).reshape(EPAD // CH, CH)
    ea_p = jnp.zeros((EPAD, 8), f32).at[:E, :4].set(edge_attr)
    x_p = jnp.zeros((N, D), f32).at[:, :3].set(x)
    zeros_nd = jnp.zeros((N, D), f32)
    ones_e = jnp.zeros((EPAD, D), f32).at[:E, 0].set(1.0)
    batch8 = jnp.tile(batch[:, None], (1, 8))

    cparts = _sc_scatter(ones_e, dst_p, zeros_nd)
    c0, c1 = cparts[:N], cparts[N:]

    def layer_weights(i, c_real):
        w1 = p["nn%d_w1" % i]
        w2 = p["nn%d_w2" % i]
        w1t = jnp.zeros((8, 128), f32).at[:4, :].set(w1.T)
        b1 = p["nn%d_b1" % i][None, :]
        v = w2.reshape(c_real, HID, 128).transpose(0, 2, 1).reshape(
            c_real * 128, HID).astype(jnp.bfloat16)
        b2 = jnp.zeros((D, HID), f32).at[:c_real, :].set(
            p["nn%d_b2" % i].reshape(c_real, HID))
        rw = p["root%d" % i]
        rt = jnp.zeros((D, HID), f32).at[:rw.shape[1], :].set(rw.T)
        cb = p["cbias%d" % i][None, :]
        g = p["ln%d_g" % i][None, :]
        b = p["ln%d_b" % i][None, :]
        return w1t, b1, v, b2, rt, cb, g, b

    h = x_p
    for i, (c_real, act) in enumerate([(3, "relu"), (HID, "elu"), (HID, "leaky")],
                                      start=1):
        w1t, b1, v, b2, rt, cb, g, b = layer_weights(i, c_real)
        xs = _sc_gather(h, src_p)
        msg = _make_msg(c_real)(ea_p, xs, w1t, b1, v, b2)
        parts = _sc_scatter(msg, dst_p, zeros_nd)
        h = _make_pointwise(act)(parts[:N], parts[N:], c0, c1, h, rt, cb, g, b)

    lw = p["lin_w"]
    out = _pool(h, batch8, contingency_type, load_scale[:, None],
                lw[:, :HID].T, lw[:, HID:2 * HID].T, lw[:, 2 * HID:2 * HID + 4].T,
                lw[:, 2 * HID + 4:].T, p["lin_b"][None, :])
    return out


# speedup vs baseline: 1.4794x; 1.4794x over previous
import functools

import jax
import jax.numpy as jnp
from jax import lax
from jax.experimental import pallas as pl
from jax.experimental.pallas import tpu as pltpu
from jax.experimental.pallas import tpu_sc as plsc

N = 10000
E = 160000
HID = 32
NGRAPH = 64

NC = 2
NS = 16
NW = NC * NS
CH = 128
NCH = 40
EW = CH * NCH
EPAD = NW * EW
D = 32

@functools.lru_cache(maxsize=None)
def _sc_mesh():
    return plsc.VectorSubcoreMesh(core_axis_name="c", subcore_axis_name="s",
                                  num_cores=NC, num_subcores=NS)


def _gather_body(x_hbm, idx_hbm, out_hbm, idx_v, buf0, buf1, gs0, gs1):
    cid = lax.axis_index("c")
    sid = lax.axis_index("s")
    wid = sid * NC + cid
    base = wid * EW
    pltpu.sync_copy(idx_hbm.at[pl.ds(wid * NCH, NCH)], idx_v)
    pltpu.async_copy(x_hbm.at[idx_v.at[0]], buf0, gs0)

    def step(t, carry):
        j = 2 * t
        pltpu.async_copy(x_hbm.at[idx_v.at[j + 1]], buf1, gs1)
        pltpu.make_async_copy(x_hbm.at[idx_v.at[j]], buf0, gs0).wait()
        pltpu.sync_copy(buf0, out_hbm.at[pl.ds(base + j * CH, CH)])

        @pl.when(j + 2 < NCH)
        def _():
            pltpu.async_copy(x_hbm.at[idx_v.at[j + 2]], buf0, gs0)

        pltpu.make_async_copy(x_hbm.at[idx_v.at[j + 1]], buf1, gs1).wait()
        pltpu.sync_copy(buf1, out_hbm.at[pl.ds(base + (j + 1) * CH, CH)])
        return carry

    lax.fori_loop(0, NCH // 2, step, 0)


@functools.lru_cache(maxsize=None)
def _gather_call():
    return pl.kernel(
        _gather_body,
        out_type=jax.ShapeDtypeStruct((EPAD, D), jnp.float32),
        mesh=_sc_mesh(),
        compiler_params=pltpu.CompilerParams(use_tc_tiling_on_sc=False),
        scratch_types=[
            pltpu.VMEM((NCH, CH), jnp.int32),
            pltpu.VMEM((CH, D), jnp.float32),
            pltpu.VMEM((CH, D), jnp.float32),
            pltpu.SemaphoreType.DMA,
            pltpu.SemaphoreType.DMA,
        ],
    )


def _sc_gather(x, idx):
    return _gather_call()(x, idx)


def _scatter_body(with_cnt, msg_hbm, idx_hbm, zero_hbm, ones_hbm, *rest):
    if with_cnt:
        (out_hbm, cout_hbm, idx_v, buf0, buf1, ones_v, acc_sh, cacc_sh,
         gs0, gs1) = rest
    else:
        out_hbm, idx_v, buf0, buf1, acc_sh, gs0, gs1 = rest
    cid = lax.axis_index("c")
    sid = lax.axis_index("s")
    wid = sid * NC + cid
    base = wid * EW
    rb = N // NS
    pltpu.sync_copy(zero_hbm.at[pl.ds(sid * rb, rb)],
                    acc_sh.at[pl.ds(sid * rb, rb)])
    if with_cnt:
        pltpu.sync_copy(zero_hbm.at[pl.ds(sid * rb, rb)],
                        cacc_sh.at[pl.ds(sid * rb, rb)])
        pltpu.sync_copy(ones_hbm, ones_v)
    plsc.subcore_barrier()
    pltpu.sync_copy(idx_hbm.at[pl.ds(wid * NCH, NCH)], idx_v)
    pltpu.async_copy(msg_hbm.at[pl.ds(base, CH)], buf0, gs0)

    def add_one(j, buf):
        pltpu.sync_copy(buf, acc_sh.at[idx_v.at[j]], add=True)
        if with_cnt:
            @pl.when(base + j * CH < E)
            def _():
                pltpu.sync_copy(ones_v, cacc_sh.at[idx_v.at[j]], add=True)

    def step(t, carry):
        j = 2 * t
        pltpu.async_copy(msg_hbm.at[pl.ds(base + (j + 1) * CH, CH)], buf1, gs1)
        pltpu.make_async_copy(msg_hbm.at[pl.ds(base + j * CH, CH)], buf0,
                              gs0).wait()
        add_one(j, buf0)

        @pl.when(j + 2 < NCH)
        def _():
            pltpu.async_copy(msg_hbm.at[pl.ds(base + (j + 2) * CH, CH)], buf0,
                             gs0)

        pltpu.make_async_copy(msg_hbm.at[pl.ds(base + (j + 1) * CH, CH)], buf1,
                              gs1).wait()
        add_one(j + 1, buf1)
        return carry

    lax.fori_loop(0, NCH // 2, step, 0)
    plsc.subcore_barrier()
    pltpu.sync_copy(acc_sh.at[pl.ds(sid * rb, rb)],
                    out_hbm.at[pl.ds(cid * N + sid * rb, rb)])
    if with_cnt:
        pltpu.sync_copy(cacc_sh.at[pl.ds(sid * rb, rb)],
                        cout_hbm.at[pl.ds(cid * N + sid * rb, rb)])


@functools.lru_cache(maxsize=None)
def _scatter_call(with_cnt):
    out_t = jax.ShapeDtypeStruct((NC * N, D), jnp.float32)
    scratch = [
        pltpu.VMEM((NCH, CH), jnp.int32),
        pltpu.VMEM((CH, D), jnp.float32),
        pltpu.VMEM((CH, D), jnp.float32),
    ]
    if with_cnt:
        scratch += [pltpu.VMEM((CH, D), jnp.float32)]
    scratch += [pltpu.VMEM_SHARED((N, D), jnp.float32)]
    if with_cnt:
        scratch += [pltpu.VMEM_SHARED((N, D), jnp.float32)]
    scratch += [pltpu.SemaphoreType.DMA, pltpu.SemaphoreType.DMA]
    return pl.kernel(
        functools.partial(_scatter_body, with_cnt),
        out_type=(out_t, out_t) if with_cnt else out_t,
        mesh=_sc_mesh(),
        compiler_params=pltpu.CompilerParams(use_tc_tiling_on_sc=False),
        scratch_types=scratch,
    )


def _sc_scatter(msg, idx, zeros, ones=None):
    if ones is None:
        return _scatter_call(False)(msg, idx, zeros, zeros[:CH])
    return _scatter_call(True)(msg, idx, zeros, ones)


BE = 512


def _msg_body(c_real, ea_ref, xs_ref, w1t_ref, b1_ref, vp_ref, r_ref, s_ref,
              b2_ref, out_ref):
    pid = pl.program_id(0)
    h = jnp.dot(ea_ref[...], w1t_ref[...], preferred_element_type=jnp.float32)
    h = jnp.maximum(h + b1_ref[...], 0.0)
    xs = xs_ref[...]
    theta = jnp.dot(h.astype(jnp.bfloat16), vp_ref[...],
                    preferred_element_type=jnp.float32)
    til = jnp.dot(xs, r_ref[...], preferred_element_type=jnp.float32)
    p = til * theta
    msg = jnp.dot(p, s_ref[...], preferred_element_type=jnp.float32)
    msg = msg + jnp.dot(xs, b2_ref[...], preferred_element_type=jnp.float32)
    gid = pid * BE + lax.broadcasted_iota(jnp.int32, (BE, HID), 0)
    out_ref[...] = jnp.where(gid < E, msg, 0.0)


def _make_msg(c_real):
    w = c_real * HID
    return pl.pallas_call(
        functools.partial(_msg_body, c_real),
        grid=(EPAD // BE,),
        in_specs=[
            pl.BlockSpec((BE, 8), lambda i: (i, 0)),
            pl.BlockSpec((BE, D), lambda i: (i, 0)),
            pl.BlockSpec((8, 128), lambda i: (0, 0)),
            pl.BlockSpec((1, 128), lambda i: (0, 0)),
            pl.BlockSpec((128, w), lambda i: (0, 0)),
            pl.BlockSpec((D, w), lambda i: (0, 0)),
            pl.BlockSpec((w, HID), lambda i: (0, 0)),
            pl.BlockSpec((D, HID), lambda i: (0, 0)),
        ],
        out_specs=pl.BlockSpec((BE, HID), lambda i: (i, 0)),
        out_shape=jax.ShapeDtypeStruct((EPAD, HID), jnp.float32),
    )


BN = 2000


def _pointwise_body(act, p0, p1, c0, c1, x_ref, rt_ref, cb_ref, g_ref, b_ref, out_ref):
    s = p0[...] + p1[...]
    cnt = c0[...][:, :1] + c1[...][:, :1]
    mean = s / jnp.maximum(cnt, 1.0)
    r = jnp.dot(x_ref[...], rt_ref[...], preferred_element_type=jnp.float32)
    t = mean + r + cb_ref[...]
    m = jnp.mean(t, axis=1, keepdims=True)
    d = t - m
    v = jnp.mean(d * d, axis=1, keepdims=True)
    y = d * lax.rsqrt(v + 1e-5) * g_ref[...] + b_ref[...]
    if act == "relu":
        y = jnp.maximum(y, 0.0)
    elif act == "elu":
        y = jnp.where(y > 0.0, y, jnp.exp(jnp.minimum(y, 0.0)) - 1.0)
    else:
        y = jnp.where(y > 0.0, y, 0.01 * y)
    out_ref[...] = y


def _make_pointwise(act):
    blk = lambda shape: pl.BlockSpec(shape, lambda i: (i, 0))
    full = lambda shape: pl.BlockSpec(shape, lambda i: (0, 0))
    return pl.pallas_call(
        functools.partial(_pointwise_body, act),
        grid=(N // BN,),
        in_specs=[
            blk((BN, HID)), blk((BN, HID)), blk((BN, D)), blk((BN, D)),
            blk((BN, D)), full((D, HID)), full((1, HID)), full((1, HID)),
            full((1, HID)),
        ],
        out_specs=blk((BN, HID)),
        out_shape=jax.ShapeDtypeStruct((N, HID), jnp.float32),
    )


BP = 400


def _pool_body(h_ref, bt_ref, ct_ref, ld_ref, wm_ref, wx_ref, wc_ref, wl_ref,
               lb_ref, out_ref, acc_s, acc_c, acc_m):
    pid = pl.program_id(0)

    @pl.when(pid == 0)
    def _():
        acc_s[...] = jnp.zeros((NGRAPH, HID), jnp.float32)
        acc_c[...] = jnp.zeros((NGRAPH, HID), jnp.float32)
        acc_m[...] = jnp.full((NGRAPH, HID), -jnp.inf, jnp.float32)

    h = h_ref[...]
    bt = bt_ref[...][:, :1]
    ohb = bt == lax.broadcasted_iota(jnp.int32, (BP, NGRAPH), 1)
    oh = ohb.astype(jnp.float32)
    dn = (((0,), (0,)), ((), ()))
    acc_s[...] += lax.dot_general(oh, h, dn, preferred_element_type=jnp.float32)
    acc_c[...] += lax.dot_general(oh, jnp.ones((BP, HID), jnp.float32), dn,
                                  preferred_element_type=jnp.float32)
    mx = jnp.concatenate(
        [jnp.max(jnp.where(bt == g, h, -jnp.inf), axis=0, keepdims=True)
         for g in range(NGRAPH)], axis=0)
    acc_m[...] = jnp.maximum(acc_m[...], mx)

    mean = acc_s[...] / jnp.maximum(acc_c[...], 1.0)
    out = jnp.dot(mean, wm_ref[...], preferred_element_type=jnp.float32)
    out += jnp.dot(acc_m[...], wx_ref[...], preferred_element_type=jnp.float32)
    out += jnp.dot(ct_ref[...], wc_ref[...], preferred_element_type=jnp.float32)
    out += jnp.dot(ld_ref[...], wl_ref[...], preferred_element_type=jnp.float32)
    out_ref[...] = out + lb_ref[...]


_pool = pl.pallas_call(
    _pool_body,
    grid=(N // BP,),
    in_specs=[
        pl.BlockSpec((BP, HID), lambda i: (i, 0)),
        pl.BlockSpec((BP, 8), lambda i: (i, 0)),
        pl.BlockSpec((NGRAPH, 4), lambda i: (0, 0)),
        pl.BlockSpec((NGRAPH, 1), lambda i: (0, 0)),
        pl.BlockSpec((HID, 2), lambda i: (0, 0)),
        pl.BlockSpec((HID, 2), lambda i: (0, 0)),
        pl.BlockSpec((4, 2), lambda i: (0, 0)),
        pl.BlockSpec((1, 2), lambda i: (0, 0)),
        pl.BlockSpec((1, 2), lambda i: (0, 0)),
    ],
    out_specs=pl.BlockSpec((NGRAPH, 2), lambda i: (0, 0)),
    out_shape=jax.ShapeDtypeStruct((NGRAPH, 2), jnp.float32),
    scratch_shapes=[
        pltpu.VMEM((NGRAPH, HID), jnp.float32),
        pltpu.VMEM((NGRAPH, HID), jnp.float32),
        pltpu.VMEM((NGRAPH, HID), jnp.float32),
    ],
)


def kernel(x, edge_index, edge_attr, batch, contingency_type, load_scale, params):
    p = params
    f32 = jnp.float32
    src, dst = edge_index[0], edge_index[1]
    pad = EPAD - E
    zpad = jnp.zeros((pad,), jnp.int32)
    src_p = jnp.concatenate([src, zpad]).reshape(EPAD // CH, CH)
    dst_p = jnp.concatenate([dst, zpad]).reshape(EPAD // CH, CH)
    ea_p = jnp.zeros((EPAD, 8), f32).at[:E, :4].set(edge_attr)
    x_p = jnp.zeros((N, D), f32).at[:, :3].set(x)
    zeros_nd = jnp.zeros((N, D), f32)
    ones_ch = jnp.ones((CH, D), f32)
    batch8 = jnp.tile(batch[:, None], (1, 8))

    def layer_weights(i, c_real):
        w1 = p["nn%d_w1" % i]
        w2 = p["nn%d_w2" % i]
        w = c_real * HID
        w1t = jnp.zeros((8, 128), f32).at[:4, :].set(w1.T)
        b1 = p["nn%d_b1" % i][None, :]
        vp = w2.reshape(c_real, HID, 128).transpose(2, 1, 0).reshape(
            128, w).astype(jnp.bfloat16)
        m = jnp.arange(w)
        r = (jnp.arange(D)[:, None] == (m[None, :] % c_real)).astype(f32)
        s = ((m[:, None] // c_real) == jnp.arange(HID)[None, :]).astype(f32)
        b2 = jnp.zeros((D, HID), f32).at[:c_real, :].set(
            p["nn%d_b2" % i].reshape(c_real, HID))
        rw = p["root%d" % i]
        rt = jnp.zeros((D, HID), f32).at[:rw.shape[1], :].set(rw.T)
        cb = p["cbias%d" % i][None, :]
        g = p["ln%d_g" % i][None, :]
        b = p["ln%d_b" % i][None, :]
        return w1t, b1, vp, r, s, b2, rt, cb, g, b

    h = x_p
    c0 = c1 = None
    for i, (c_real, act) in enumerate([(3, "relu"), (HID, "elu"), (HID, "leaky")],
                                      start=1):
        w1t, b1, vp, r, s, b2, rt, cb, g, b = layer_weights(i, c_real)
        xs = _sc_gather(h, src_p)
        msg = _make_msg(c_real)(ea_p, xs, w1t, b1, vp, r, s, b2)
        if i == 1:
            parts, cparts = _sc_scatter(msg, dst_p, zeros_nd, ones_ch)
            c0, c1 = cparts[:N], cparts[N:]
        else:
            parts = _sc_scatter(msg, dst_p, zeros_nd)
        h = _make_pointwise(act)(parts[:N], parts[N:], c0, c1, h, rt, cb, g, b)

    lw = p["lin_w"]
    out = _pool(h, batch8, contingency_type, load_scale[:, None],
                lw[:, :HID].T, lw[:, HID:2 * HID].T, lw[:, 2 * HID:2 * HID + 4].T,
                lw[:, 2 * HID + 4:].T, p["lin_b"][None, :])
    return out


# speedup vs baseline: 1.7382x; 1.1750x over previous
import functools

import jax
import jax.numpy as jnp
from jax import lax
from jax.experimental import pallas as pl
from jax.experimental.pallas import tpu as pltpu
from jax.experimental.pallas import tpu_sc as plsc

N = 10000
E = 160000
HID = 32
NGRAPH = 64

NC = 2
NS = 16
NW = NC * NS
CH = 128
NCH = 40
EW = CH * NCH
EPAD = NW * EW
D = 32

@functools.lru_cache(maxsize=None)
def _sc_mesh():
    return plsc.VectorSubcoreMesh(core_axis_name="c", subcore_axis_name="s",
                                  num_cores=NC, num_subcores=NS)


def _gather_body(x_hbm, idx_hbm, out_hbm, idx_v, buf0, buf1, gs0, gs1):
    cid = lax.axis_index("c")
    sid = lax.axis_index("s")
    wid = sid * NC + cid
    base = wid * EW
    pltpu.sync_copy(idx_hbm.at[pl.ds(wid * NCH, NCH)], idx_v)
    pltpu.async_copy(x_hbm.at[idx_v.at[0]], buf0, gs0)

    def step(t, carry):
        j = 2 * t
        pltpu.async_copy(x_hbm.at[idx_v.at[j + 1]], buf1, gs1)
        pltpu.make_async_copy(x_hbm.at[idx_v.at[j]], buf0, gs0).wait()
        pltpu.sync_copy(buf0, out_hbm.at[pl.ds(base + j * CH, CH)])

        @pl.when(j + 2 < NCH)
        def _():
            pltpu.async_copy(x_hbm.at[idx_v.at[j + 2]], buf0, gs0)

        pltpu.make_async_copy(x_hbm.at[idx_v.at[j + 1]], buf1, gs1).wait()
        pltpu.sync_copy(buf1, out_hbm.at[pl.ds(base + (j + 1) * CH, CH)])
        return carry

    lax.fori_loop(0, NCH // 2, step, 0)


@functools.lru_cache(maxsize=None)
def _gather_call():
    return pl.kernel(
        _gather_body,
        out_type=jax.ShapeDtypeStruct((EPAD, D), jnp.float32),
        mesh=_sc_mesh(),
        compiler_params=pltpu.CompilerParams(use_tc_tiling_on_sc=False),
        scratch_types=[
            pltpu.VMEM((NCH, CH), jnp.int32),
            pltpu.VMEM((CH, D), jnp.float32),
            pltpu.VMEM((CH, D), jnp.float32),
            pltpu.SemaphoreType.DMA,
            pltpu.SemaphoreType.DMA,
        ],
    )


def _sc_gather(x, idx):
    return _gather_call()(x, idx)


def _scatter_body(with_cnt, msg_hbm, idx_hbm, zero_hbm, ones_hbm, *rest):
    if with_cnt:
        (out_hbm, cout_hbm, idx_v, buf0, buf1, ones_v, acc_sh, cacc_sh,
         gs0, gs1) = rest
    else:
        out_hbm, idx_v, buf0, buf1, acc_sh, gs0, gs1 = rest
    cid = lax.axis_index("c")
    sid = lax.axis_index("s")
    wid = sid * NC + cid
    base = wid * EW
    rb = N // NS
    pltpu.sync_copy(zero_hbm.at[pl.ds(sid * rb, rb)],
                    acc_sh.at[pl.ds(sid * rb, rb)])
    if with_cnt:
        pltpu.sync_copy(zero_hbm.at[pl.ds(sid * rb, rb)],
                        cacc_sh.at[pl.ds(sid * rb, rb)])
        pltpu.sync_copy(ones_hbm, ones_v)
    plsc.subcore_barrier()
    pltpu.sync_copy(idx_hbm.at[pl.ds(wid * NCH, NCH)], idx_v)
    pltpu.async_copy(msg_hbm.at[pl.ds(base, CH)], buf0, gs0)

    def add_one(j, buf):
        pltpu.sync_copy(buf, acc_sh.at[idx_v.at[j]], add=True)
        if with_cnt:
            @pl.when(base + j * CH < E)
            def _():
                pltpu.sync_copy(ones_v, cacc_sh.at[idx_v.at[j]], add=True)

    def step(t, carry):
        j = 2 * t
        pltpu.async_copy(msg_hbm.at[pl.ds(base + (j + 1) * CH, CH)], buf1, gs1)
        pltpu.make_async_copy(msg_hbm.at[pl.ds(base + j * CH, CH)], buf0,
                              gs0).wait()
        add_one(j, buf0)

        @pl.when(j + 2 < NCH)
        def _():
            pltpu.async_copy(msg_hbm.at[pl.ds(base + (j + 2) * CH, CH)], buf0,
                             gs0)

        pltpu.make_async_copy(msg_hbm.at[pl.ds(base + (j + 1) * CH, CH)], buf1,
                              gs1).wait()
        add_one(j + 1, buf1)
        return carry

    lax.fori_loop(0, NCH // 2, step, 0)
    plsc.subcore_barrier()
    pltpu.sync_copy(acc_sh.at[pl.ds(sid * rb, rb)],
                    out_hbm.at[pl.ds(cid * N + sid * rb, rb)])
    if with_cnt:
        pltpu.sync_copy(cacc_sh.at[pl.ds(sid * rb, rb)],
                        cout_hbm.at[pl.ds(cid * N + sid * rb, rb)])


@functools.lru_cache(maxsize=None)
def _scatter_call(with_cnt):
    out_t = jax.ShapeDtypeStruct((NC * N, D), jnp.float32)
    scratch = [
        pltpu.VMEM((NCH, CH), jnp.int32),
        pltpu.VMEM((CH, D), jnp.float32),
        pltpu.VMEM((CH, D), jnp.float32),
    ]
    if with_cnt:
        scratch += [pltpu.VMEM((CH, D), jnp.float32)]
    scratch += [pltpu.VMEM_SHARED((N, D), jnp.float32)]
    if with_cnt:
        scratch += [pltpu.VMEM_SHARED((N, D), jnp.float32)]
    scratch += [pltpu.SemaphoreType.DMA, pltpu.SemaphoreType.DMA]
    return pl.kernel(
        functools.partial(_scatter_body, with_cnt),
        out_type=(out_t, out_t) if with_cnt else out_t,
        mesh=_sc_mesh(),
        compiler_params=pltpu.CompilerParams(use_tc_tiling_on_sc=False),
        scratch_types=scratch,
    )


def _sc_scatter(msg, idx, zeros, ones=None):
    if ones is None:
        return _scatter_call(False)(msg, idx, zeros, zeros[:CH])
    return _scatter_call(True)(msg, idx, zeros, ones)


BE = 1024


def _msg_body(c_real, ea_ref, xs_ref, w1t_ref, b1_ref, vp_ref, r_ref, s_ref,
              b2_ref, out_ref):
    pid = pl.program_id(0)
    h = jnp.dot(ea_ref[...], w1t_ref[...], preferred_element_type=jnp.float32)
    h = jnp.maximum(h + b1_ref[...], 0.0)
    xs = xs_ref[...]
    theta = jnp.dot(h.astype(jnp.bfloat16), vp_ref[...],
                    preferred_element_type=jnp.float32)
    til = jnp.dot(xs.astype(jnp.bfloat16), r_ref[...],
                  preferred_element_type=jnp.float32)
    p = (til * theta).astype(jnp.bfloat16)
    msg = jnp.dot(p, s_ref[...], preferred_element_type=jnp.float32)
    msg = msg + jnp.dot(xs, b2_ref[...], preferred_element_type=jnp.float32)
    gid = pid * BE + lax.broadcasted_iota(jnp.int32, (BE, HID), 0)
    out_ref[...] = jnp.where(gid < E, msg, 0.0)


def _make_msg(c_real):
    w = c_real * HID
    nbe = (E + BE - 1) // BE - 1
    return pl.pallas_call(
        functools.partial(_msg_body, c_real),
        grid=(EPAD // BE,),
        in_specs=[
            pl.BlockSpec((BE, 4), lambda i: (jnp.minimum(i, nbe), 0)),
            pl.BlockSpec((BE, D), lambda i: (i, 0)),
            pl.BlockSpec((4, 128), lambda i: (0, 0)),
            pl.BlockSpec((1, 128), lambda i: (0, 0)),
            pl.BlockSpec((128, w), lambda i: (0, 0)),
            pl.BlockSpec((D, w), lambda i: (0, 0)),
            pl.BlockSpec((w, HID), lambda i: (0, 0)),
            pl.BlockSpec((D, HID), lambda i: (0, 0)),
        ],
        out_specs=pl.BlockSpec((BE, HID), lambda i: (i, 0)),
        out_shape=jax.ShapeDtypeStruct((EPAD, HID), jnp.float32),
    )


BN = 2000


def _pointwise_body(act, p0, p1, c0, c1, x_ref, rt_ref, cb_ref, g_ref, b_ref, out_ref):
    s = p0[...] + p1[...]
    cnt = c0[...][:, :1] + c1[...][:, :1]
    mean = s / jnp.maximum(cnt, 1.0)
    r = jnp.dot(x_ref[...], rt_ref[...], preferred_element_type=jnp.float32)
    t = mean + r + cb_ref[...]
    m = jnp.mean(t, axis=1, keepdims=True)
    d = t - m
    v = jnp.mean(d * d, axis=1, keepdims=True)
    y = d * lax.rsqrt(v + 1e-5) * g_ref[...] + b_ref[...]
    if act == "relu":
        y = jnp.maximum(y, 0.0)
    elif act == "elu":
        y = jnp.where(y > 0.0, y, jnp.exp(jnp.minimum(y, 0.0)) - 1.0)
    else:
        y = jnp.where(y > 0.0, y, 0.01 * y)
    out_ref[...] = y


def _make_pointwise(act):
    blk = lambda shape: pl.BlockSpec(shape, lambda i: (i, 0))
    full = lambda shape: pl.BlockSpec(shape, lambda i: (0, 0))
    return pl.pallas_call(
        functools.partial(_pointwise_body, act),
        grid=(N // BN,),
        in_specs=[
            blk((BN, HID)), blk((BN, HID)), blk((BN, D)), blk((BN, D)),
            blk((BN, D)), full((D, HID)), full((1, HID)), full((1, HID)),
            full((1, HID)),
        ],
        out_specs=blk((BN, HID)),
        out_shape=jax.ShapeDtypeStruct((N, HID), jnp.float32),
    )


BP = 400


def _pool_body(h_ref, bt_ref, ct_ref, ld_ref, wm_ref, wx_ref, wc_ref, wl_ref,
               lb_ref, out_ref, acc_s, acc_c, acc_m):
    pid = pl.program_id(0)

    @pl.when(pid == 0)
    def _():
        acc_s[...] = jnp.zeros((NGRAPH, HID), jnp.float32)
        acc_c[...] = jnp.zeros((NGRAPH, HID), jnp.float32)
        acc_m[...] = jnp.full((NGRAPH, HID), -jnp.inf, jnp.float32)

    h = h_ref[...]
    bt = bt_ref[...][:, :1]
    ohb = bt == lax.broadcasted_iota(jnp.int32, (BP, NGRAPH), 1)
    oh = ohb.astype(jnp.float32)
    dn = (((0,), (0,)), ((), ()))
    acc_s[...] += lax.dot_general(oh, h, dn, preferred_element_type=jnp.float32)
    acc_c[...] += lax.dot_general(oh, jnp.ones((BP, HID), jnp.float32), dn,
                                  preferred_element_type=jnp.float32)
    mx = jnp.concatenate(
        [jnp.max(jnp.where(bt == g, h, -jnp.inf), axis=0, keepdims=True)
         for g in range(NGRAPH)], axis=0)
    acc_m[...] = jnp.maximum(acc_m[...], mx)

    mean = acc_s[...] / jnp.maximum(acc_c[...], 1.0)
    out = jnp.dot(mean, wm_ref[...], preferred_element_type=jnp.float32)
    out += jnp.dot(acc_m[...], wx_ref[...], preferred_element_type=jnp.float32)
    out += jnp.dot(ct_ref[...], wc_ref[...], preferred_element_type=jnp.float32)
    out += jnp.dot(ld_ref[...], wl_ref[...], preferred_element_type=jnp.float32)
    out_ref[...] = out + lb_ref[...]


_pool = pl.pallas_call(
    _pool_body,
    grid=(N // BP,),
    in_specs=[
        pl.BlockSpec((BP, HID), lambda i: (i, 0)),
        pl.BlockSpec((BP, 8), lambda i: (i, 0)),
        pl.BlockSpec((NGRAPH, 4), lambda i: (0, 0)),
        pl.BlockSpec((NGRAPH, 1), lambda i: (0, 0)),
        pl.BlockSpec((HID, 2), lambda i: (0, 0)),
        pl.BlockSpec((HID, 2), lambda i: (0, 0)),
        pl.BlockSpec((4, 2), lambda i: (0, 0)),
        pl.BlockSpec((1, 2), lambda i: (0, 0)),
        pl.BlockSpec((1, 2), lambda i: (0, 0)),
    ],
    out_specs=pl.BlockSpec((NGRAPH, 2), lambda i: (0, 0)),
    out_shape=jax.ShapeDtypeStruct((NGRAPH, 2), jnp.float32),
    scratch_shapes=[
        pltpu.VMEM((NGRAPH, HID), jnp.float32),
        pltpu.VMEM((NGRAPH, HID), jnp.float32),
        pltpu.VMEM((NGRAPH, HID), jnp.float32),
    ],
)


def kernel(x, edge_index, edge_attr, batch, contingency_type, load_scale, params):
    p = params
    f32 = jnp.float32
    src, dst = edge_index[0], edge_index[1]
    pad = EPAD - E
    zpad = jnp.zeros((pad,), jnp.int32)
    src_p = jnp.concatenate([src, zpad]).reshape(EPAD // CH, CH)
    dst_p = jnp.concatenate([dst, zpad]).reshape(EPAD // CH, CH)
    x_p = jnp.zeros((N, D), f32).at[:, :3].set(x)
    zeros_nd = jnp.zeros((N, D), f32)
    ones_ch = jnp.ones((CH, D), f32)
    batch8 = jnp.tile(batch[:, None], (1, 8))

    def layer_weights(i, c_real):
        w1 = p["nn%d_w1" % i]
        w2 = p["nn%d_w2" % i]
        w = c_real * HID
        w1t = w1.T
        b1 = p["nn%d_b1" % i][None, :]
        vp = w2.reshape(c_real, HID, 128).transpose(2, 1, 0).reshape(
            128, w).astype(jnp.bfloat16)
        m = jnp.arange(w)
        r = (jnp.arange(D)[:, None] == (m[None, :] % c_real)).astype(jnp.bfloat16)
        s = ((m[:, None] // c_real) == jnp.arange(HID)[None, :]).astype(
            jnp.bfloat16)
        b2 = jnp.zeros((D, HID), f32).at[:c_real, :].set(
            p["nn%d_b2" % i].reshape(c_real, HID))
        rw = p["root%d" % i]
        rt = jnp.zeros((D, HID), f32).at[:rw.shape[1], :].set(rw.T)
        cb = p["cbias%d" % i][None, :]
        g = p["ln%d_g" % i][None, :]
        b = p["ln%d_b" % i][None, :]
        return w1t, b1, vp, r, s, b2, rt, cb, g, b

    h = x_p
    c0 = c1 = None
    for i, (c_real, act) in enumerate([(3, "relu"), (HID, "elu"), (HID, "leaky")],
                                      start=1):
        w1t, b1, vp, r, s, b2, rt, cb, g, b = layer_weights(i, c_real)
        xs = _sc_gather(h, src_p)
        msg = _make_msg(c_real)(edge_attr, xs, w1t, b1, vp, r, s, b2)
        if i == 1:
            parts, cparts = _sc_scatter(msg, dst_p, zeros_nd, ones_ch)
            c0, c1 = cparts[:N], cparts[N:]
        else:
            parts = _sc_scatter(msg, dst_p, zeros_nd)
        h = _make_pointwise(act)(parts[:N], parts[N:], c0, c1, h, rt, cb, g, b)

    lw = p["lin_w"]
    out = _pool(h, batch8, contingency_type, load_scale[:, None],
                lw[:, :HID].T, lw[:, HID:2 * HID].T, lw[:, 2 * HID:2 * HID + 4].T,
                lw[:, 2 * HID + 4:].T, p["lin_b"][None, :])
    return out


# speedup vs baseline: 1.7874x; 1.0283x over previous
import functools

import jax
import jax.numpy as jnp
from jax import lax
from jax.experimental import pallas as pl
from jax.experimental.pallas import tpu as pltpu
from jax.experimental.pallas import tpu_sc as plsc

N = 10000
E = 160000
HID = 32
NGRAPH = 64

NC = 2
NS = 16
NW = NC * NS
CH = 128
D = 32

NHALF = 2
EH = E // NHALF
NCH = 20
EW = CH * NCH
EPADH = NW * EW


@functools.lru_cache(maxsize=None)
def _sc_mesh():
    return plsc.VectorSubcoreMesh(core_axis_name="c", subcore_axis_name="s",
                                  num_cores=NC, num_subcores=NS)


def _gather_body(x_hbm, idx_hbm, out_hbm, idx_v, buf0, buf1, gs0, gs1):
    cid = lax.axis_index("c")
    sid = lax.axis_index("s")
    wid = sid * NC + cid
    base = wid * EW
    pltpu.sync_copy(idx_hbm.at[pl.ds(wid * NCH, NCH)], idx_v)
    pltpu.async_copy(x_hbm.at[idx_v.at[0]], buf0, gs0)

    def step(t, carry):
        j = 2 * t
        pltpu.async_copy(x_hbm.at[idx_v.at[j + 1]], buf1, gs1)
        pltpu.make_async_copy(x_hbm.at[idx_v.at[j]], buf0, gs0).wait()
        pltpu.sync_copy(buf0, out_hbm.at[pl.ds(base + j * CH, CH)])

        @pl.when(j + 2 < NCH)
        def _():
            pltpu.async_copy(x_hbm.at[idx_v.at[j + 2]], buf0, gs0)

        pltpu.make_async_copy(x_hbm.at[idx_v.at[j + 1]], buf1, gs1).wait()
        pltpu.sync_copy(buf1, out_hbm.at[pl.ds(base + (j + 1) * CH, CH)])
        return carry

    lax.fori_loop(0, NCH // 2, step, 0)


@functools.lru_cache(maxsize=None)
def _gather_call():
    return pl.kernel(
        _gather_body,
        out_type=jax.ShapeDtypeStruct((EPADH, D), jnp.float32),
        mesh=_sc_mesh(),
        compiler_params=pltpu.CompilerParams(use_tc_tiling_on_sc=False),
        scratch_types=[
            pltpu.VMEM((NCH, CH), jnp.int32),
            pltpu.VMEM((CH, D), jnp.float32),
            pltpu.VMEM((CH, D), jnp.float32),
            pltpu.SemaphoreType.DMA,
            pltpu.SemaphoreType.DMA,
        ],
    )


def _sc_gather(x, idx):
    return _gather_call()(x, idx)


def _scatter_body(with_cnt, msg_hbm, idx_hbm, zero_hbm, ones_hbm, *rest):
    if with_cnt:
        (out_hbm, cout_hbm, idx_v, buf0, buf1, ones_v, acc_sh, cacc_sh,
         gs0, gs1) = rest
    else:
        out_hbm, idx_v, buf0, buf1, acc_sh, gs0, gs1 = rest
    cid = lax.axis_index("c")
    sid = lax.axis_index("s")
    wid = sid * NC + cid
    base = wid * EW
    rb = N // NS
    pltpu.sync_copy(zero_hbm.at[pl.ds(sid * rb, rb)],
                    acc_sh.at[pl.ds(sid * rb, rb)])
    if with_cnt:
        pltpu.sync_copy(zero_hbm.at[pl.ds(sid * rb, rb)],
                        cacc_sh.at[pl.ds(sid * rb, rb)])
        pltpu.sync_copy(ones_hbm, ones_v)
    plsc.subcore_barrier()
    pltpu.sync_copy(idx_hbm.at[pl.ds(wid * NCH, NCH)], idx_v)
    pltpu.async_copy(msg_hbm.at[pl.ds(base, CH)], buf0, gs0)

    def add_one(j, buf):
        pltpu.sync_copy(buf, acc_sh.at[idx_v.at[j]], add=True)
        if with_cnt:
            @pl.when(base + j * CH < EH)
            def _():
                pltpu.sync_copy(ones_v, cacc_sh.at[idx_v.at[j]], add=True)

    def step(t, carry):
        j = 2 * t
        pltpu.async_copy(msg_hbm.at[pl.ds(base + (j + 1) * CH, CH)], buf1, gs1)
        pltpu.make_async_copy(msg_hbm.at[pl.ds(base + j * CH, CH)], buf0,
                              gs0).wait()
        add_one(j, buf0)

        @pl.when(j + 2 < NCH)
        def _():
            pltpu.async_copy(msg_hbm.at[pl.ds(base + (j + 2) * CH, CH)], buf0,
                             gs0)

        pltpu.make_async_copy(msg_hbm.at[pl.ds(base + (j + 1) * CH, CH)], buf1,
                              gs1).wait()
        add_one(j + 1, buf1)
        return carry

    lax.fori_loop(0, NCH // 2, step, 0)
    plsc.subcore_barrier()
    pltpu.sync_copy(acc_sh.at[pl.ds(sid * rb, rb)],
                    out_hbm.at[pl.ds(cid * N + sid * rb, rb)])
    if with_cnt:
        pltpu.sync_copy(cacc_sh.at[pl.ds(sid * rb, rb)],
                        cout_hbm.at[pl.ds(cid * N + sid * rb, rb)])


@functools.lru_cache(maxsize=None)
def _scatter_call(with_cnt):
    out_t = jax.ShapeDtypeStruct((NC * N, D), jnp.float32)
    scratch = [
        pltpu.VMEM((NCH, CH), jnp.int32),
        pltpu.VMEM((CH, D), jnp.float32),
        pltpu.VMEM((CH, D), jnp.float32),
    ]
    if with_cnt:
        scratch += [pltpu.VMEM((CH, D), jnp.float32)]
    scratch += [pltpu.VMEM_SHARED((N, D), jnp.float32)]
    if with_cnt:
        scratch += [pltpu.VMEM_SHARED((N, D), jnp.float32)]
    scratch += [pltpu.SemaphoreType.DMA, pltpu.SemaphoreType.DMA]
    return pl.kernel(
        functools.partial(_scatter_body, with_cnt),
        out_type=(out_t, out_t) if with_cnt else out_t,
        mesh=_sc_mesh(),
        compiler_params=pltpu.CompilerParams(use_tc_tiling_on_sc=False),
        scratch_types=scratch,
    )


def _sc_scatter(msg, idx, zeros, ones=None):
    if ones is None:
        return _scatter_call(False)(msg, idx, zeros, zeros[:CH])
    return _scatter_call(True)(msg, idx, zeros, ones)


BE = 1024


def _msg_body(c_real, ea_ref, xs_ref, w1t_ref, b1_ref, vp_ref, r_ref, s_ref,
              b2_ref, out_ref):
    pid = pl.program_id(0)
    h = jnp.dot(ea_ref[...], w1t_ref[...], preferred_element_type=jnp.float32)
    h = jnp.maximum(h + b1_ref[...], 0.0)
    xs = xs_ref[...]
    theta = jnp.dot(h.astype(jnp.bfloat16), vp_ref[...],
                    preferred_element_type=jnp.float32)
    til = jnp.dot(xs.astype(jnp.bfloat16), r_ref[...],
                  preferred_element_type=jnp.float32)
    p = (til * theta).astype(jnp.bfloat16)
    msg = jnp.dot(p, s_ref[...], preferred_element_type=jnp.float32)
    msg = msg + jnp.dot(xs, b2_ref[...], preferred_element_type=jnp.float32)
    gid = pid * BE + lax.broadcasted_iota(jnp.int32, (BE, HID), 0)
    out_ref[...] = jnp.where(gid < EH, msg, 0.0)


def _make_msg(c_real):
    w = c_real * HID
    nbe = (EH + BE - 1) // BE - 1
    return pl.pallas_call(
        functools.partial(_msg_body, c_real),
        grid=(EPADH // BE,),
        in_specs=[
            pl.BlockSpec((BE, 4), lambda i: (jnp.minimum(i, nbe), 0)),
            pl.BlockSpec((BE, D), lambda i: (i, 0)),
            pl.BlockSpec((4, 128), lambda i: (0, 0)),
            pl.BlockSpec((1, 128), lambda i: (0, 0)),
            pl.BlockSpec((128, w), lambda i: (0, 0)),
            pl.BlockSpec((D, w), lambda i: (0, 0)),
            pl.BlockSpec((w, HID), lambda i: (0, 0)),
            pl.BlockSpec((D, HID), lambda i: (0, 0)),
        ],
        out_specs=pl.BlockSpec((BE, HID), lambda i: (i, 0)),
        out_shape=jax.ShapeDtypeStruct((EPADH, HID), jnp.float32),
    )


BN = 2000


def _pointwise_body(act, p0, p1, p2, p3, c0, c1, c2, c3, x_ref, rt_ref,
                    cb_ref, g_ref, b_ref, out_ref):
    s = p0[...] + p1[...] + p2[...] + p3[...]
    cnt = (c0[...][:, :1] + c1[...][:, :1] + c2[...][:, :1] + c3[...][:, :1])
    mean = s / jnp.maximum(cnt, 1.0)
    r = jnp.dot(x_ref[...], rt_ref[...], preferred_element_type=jnp.float32)
    t = mean + r + cb_ref[...]
    m = jnp.mean(t, axis=1, keepdims=True)
    d = t - m
    v = jnp.mean(d * d, axis=1, keepdims=True)
    y = d * lax.rsqrt(v + 1e-5) * g_ref[...] + b_ref[...]
    if act == "relu":
        y = jnp.maximum(y, 0.0)
    elif act == "elu":
        y = jnp.where(y > 0.0, y, jnp.exp(jnp.minimum(y, 0.0)) - 1.0)
    else:
        y = jnp.where(y > 0.0, y, 0.01 * y)
    out_ref[...] = y


def _make_pointwise(act):
    blk = lambda shape: pl.BlockSpec(shape, lambda i: (i, 0))
    full = lambda shape: pl.BlockSpec(shape, lambda i: (0, 0))
    return pl.pallas_call(
        functools.partial(_pointwise_body, act),
        grid=(N // BN,),
        in_specs=[
            blk((BN, HID)), blk((BN, HID)), blk((BN, HID)), blk((BN, HID)),
            blk((BN, D)), blk((BN, D)), blk((BN, D)), blk((BN, D)),
            blk((BN, D)), full((D, HID)), full((1, HID)), full((1, HID)),
            full((1, HID)),
        ],
        out_specs=blk((BN, HID)),
        out_shape=jax.ShapeDtypeStruct((N, HID), jnp.float32),
    )


BP = 400


def _pool_body(h_ref, bt_ref, ct_ref, ld_ref, wm_ref, wx_ref, wc_ref, wl_ref,
               lb_ref, out_ref, acc_s, acc_c, acc_m):
    pid = pl.program_id(0)

    @pl.when(pid == 0)
    def _():
        acc_s[...] = jnp.zeros((NGRAPH, HID), jnp.float32)
        acc_c[...] = jnp.zeros((NGRAPH, HID), jnp.float32)
        acc_m[...] = jnp.full((NGRAPH, HID), -jnp.inf, jnp.float32)

    h = h_ref[...]
    bt = bt_ref[...][:, :1]
    ohb = bt == lax.broadcasted_iota(jnp.int32, (BP, NGRAPH), 1)
    oh = ohb.astype(jnp.float32)
    dn = (((0,), (0,)), ((), ()))
    acc_s[...] += lax.dot_general(oh, h, dn, preferred_element_type=jnp.float32)
    acc_c[...] += lax.dot_general(oh, jnp.ones((BP, HID), jnp.float32), dn,
                                  preferred_element_type=jnp.float32)
    mx = jnp.concatenate(
        [jnp.max(jnp.where(bt == g, h, -jnp.inf), axis=0, keepdims=True)
         for g in range(NGRAPH)], axis=0)
    acc_m[...] = jnp.maximum(acc_m[...], mx)

    mean = acc_s[...] / jnp.maximum(acc_c[...], 1.0)
    out = jnp.dot(mean, wm_ref[...], preferred_element_type=jnp.float32)
    out += jnp.dot(acc_m[...], wx_ref[...], preferred_element_type=jnp.float32)
    out += jnp.dot(ct_ref[...], wc_ref[...], preferred_element_type=jnp.float32)
    out += jnp.dot(ld_ref[...], wl_ref[...], preferred_element_type=jnp.float32)
    out_ref[...] = out + lb_ref[...]


_pool = pl.pallas_call(
    _pool_body,
    grid=(N // BP,),
    in_specs=[
        pl.BlockSpec((BP, HID), lambda i: (i, 0)),
        pl.BlockSpec((BP, 8), lambda i: (i, 0)),
        pl.BlockSpec((NGRAPH, 4), lambda i: (0, 0)),
        pl.BlockSpec((NGRAPH, 1), lambda i: (0, 0)),
        pl.BlockSpec((HID, 2), lambda i: (0, 0)),
        pl.BlockSpec((HID, 2), lambda i: (0, 0)),
        pl.BlockSpec((4, 2), lambda i: (0, 0)),
        pl.BlockSpec((1, 2), lambda i: (0, 0)),
        pl.BlockSpec((1, 2), lambda i: (0, 0)),
    ],
    out_specs=pl.BlockSpec((NGRAPH, 2), lambda i: (0, 0)),
    out_shape=jax.ShapeDtypeStruct((NGRAPH, 2), jnp.float32),
    scratch_shapes=[
        pltpu.VMEM((NGRAPH, HID), jnp.float32),
        pltpu.VMEM((NGRAPH, HID), jnp.float32),
        pltpu.VMEM((NGRAPH, HID), jnp.float32),
    ],
)


def kernel(x, edge_index, edge_attr, batch, contingency_type, load_scale, params):
    p = params
    f32 = jnp.float32
    src, dst = edge_index[0], edge_index[1]
    pad = EPADH - EH
    zpad = jnp.zeros((pad,), jnp.int32)

    def half_idx(a):
        return a.reshape(EPADH // CH, CH)

    src_h = [half_idx(jnp.concatenate([src[k * EH:(k + 1) * EH], zpad]))
             for k in range(NHALF)]
    dst_h = [half_idx(jnp.concatenate([dst[k * EH:(k + 1) * EH], zpad]))
             for k in range(NHALF)]
    ea_h = [edge_attr[k * EH:(k + 1) * EH] for k in range(NHALF)]
    x_p = jnp.zeros((N, D), f32).at[:, :3].set(x)
    zeros_nd = jnp.zeros((N, D), f32)
    ones_ch = jnp.ones((CH, D), f32)
    batch8 = jnp.tile(batch[:, None], (1, 8))

    def layer_weights(i, c_real):
        w1 = p["nn%d_w1" % i]
        w2 = p["nn%d_w2" % i]
        w = c_real * HID
        w1t = w1.T
        b1 = p["nn%d_b1" % i][None, :]
        vp = w2.reshape(c_real, HID, 128).transpose(2, 1, 0).reshape(
            128, w).astype(jnp.bfloat16)
        m = jnp.arange(w)
        r = (jnp.arange(D)[:, None] == (m[None, :] % c_real)).astype(jnp.bfloat16)
        s = ((m[:, None] // c_real) == jnp.arange(HID)[None, :]).astype(
            jnp.bfloat16)
        b2 = jnp.zeros((D, HID), f32).at[:c_real, :].set(
            p["nn%d_b2" % i].reshape(c_real, HID))
        rw = p["root%d" % i]
        rt = jnp.zeros((D, HID), f32).at[:rw.shape[1], :].set(rw.T)
        cb = p["cbias%d" % i][None, :]
        g = p["ln%d_g" % i][None, :]
        b = p["ln%d_b" % i][None, :]
        return w1t, b1, vp, r, s, b2, rt, cb, g, b

    h = x_p
    cs = None
    for i, (c_real, act) in enumerate([(3, "relu"), (HID, "elu"), (HID, "leaky")],
                                      start=1):
        w1t, b1, vp, r, s, b2, rt, cb, g, b = layer_weights(i, c_real)
        mk = _make_msg(c_real)
        xs = [_sc_gather(h, src_h[k]) for k in range(NHALF)]
        msg = [mk(ea_h[k], xs[k], w1t, b1, vp, r, s, b2) for k in range(NHALF)]
        ps = []
        if i == 1:
            cs = []
            for k in range(NHALF):
                pk, ck = _sc_scatter(msg[k], dst_h[k], zeros_nd, ones_ch)
                ps += [pk[:N], pk[N:]]
                cs += [ck[:N], ck[N:]]
        else:
            for k in range(NHALF):
                pk = _sc_scatter(msg[k], dst_h[k], zeros_nd)
                ps += [pk[:N], pk[N:]]
        h = _make_pointwise(act)(*ps, *cs, h, rt, cb, g, b)

    lw = p["lin_w"]
    out = _pool(h, batch8, contingency_type, load_scale[:, None],
                lw[:, :HID].T, lw[:, HID:2 * HID].T, lw[:, 2 * HID:2 * HID + 4].T,
                lw[:, 2 * HID + 4:].T, p["lin_b"][None, :])
    return out
